# Initial kernel scaffold; baseline (speedup 1.0000x reference)
import jax
import jax.numpy as jnp
from jax.experimental import pallas as pl


def kernel(x, W, b):
    raise NotImplementedError("write your pallas kernel here")


# speedup vs baseline: 5.2275x; 5.2275x over previous
import functools

import jax
import jax.numpy as jnp
from jax.experimental import pallas as pl
from jax.experimental.pallas import tpu as pltpu

TOPK = 8


def _gating_block(x_ref, w_ref, b_ref, o_ref):
    logits = jax.lax.dot_general(
        x_ref[...], w_ref[...],
        (((1,), (1,)), ((), ())),
        preferred_element_type=jnp.float32,
    ) + b_ref[...]
    m = jnp.max(logits, axis=-1, keepdims=True)
    e = jnp.exp(logits - m)
    probs = e / jnp.sum(e, axis=-1, keepdims=True)

    n_exp = probs.shape[-1]
    colid = jax.lax.broadcasted_iota(jnp.int32, probs.shape, 1)
    work = probs
    out = jnp.zeros_like(probs)
    for _ in range(TOPK):
        mx = jnp.max(work, axis=-1, keepdims=True)
        ismax = work == mx
        first = jnp.min(jnp.where(ismax, colid, n_exp), axis=-1, keepdims=True)
        sel = colid == first
        out = jnp.where(sel, probs, out)
        work = jnp.where(sel, -1.0, work)
    o_ref[...] = out


@functools.partial(jax.jit, static_argnames=("block_rows",))
def _gating_tc(x, W, b, block_rows=512):
    n, d = x.shape
    n_exp = W.shape[0]
    grid = (n // block_rows,)
    return pl.pallas_call(
        _gating_block,
        grid=grid,
        in_specs=[
            pl.BlockSpec((block_rows, d), lambda i: (i, 0)),
            pl.BlockSpec((n_exp, d), lambda i: (0, 0)),
            pl.BlockSpec((1, n_exp), lambda i: (0, 0)),
        ],
        out_specs=pl.BlockSpec((block_rows, n_exp), lambda i: (i, 0)),
        out_shape=jax.ShapeDtypeStruct((n, n_exp), jnp.float32),
    )(x, W, b.reshape(1, n_exp))


def kernel(x, W, b):
    return _gating_tc(x, W, b)
